# baseline (speedup 1.0000x reference)
import functools

import jax
import jax.numpy as jnp
import numpy as np
from jax import lax
from jax.experimental import pallas as pl
from jax.experimental.pallas import tpu as pltpu
from jax.experimental.pallas import tpu_sc as plsc

B = 16384
D = 128
L = 16
NC = 2
NS = 16
NW = NC * NS
ROWS_PER_W = B // NW
BLK = 128
NBLK = ROWS_PER_W // BLK
NBUF = 2

@functools.cache
def _build():
    mesh = plsc.VectorSubcoreMesh(
        core_axis_name="c", subcore_axis_name="s", num_cores=NC, num_subcores=NS
    )

    @functools.partial(
        pl.kernel,
        out_type=jax.ShapeDtypeStruct((B,), jnp.float32),
        mesh=mesh,
        scratch_types=[
            pltpu.VMEM((NBUF, BLK, D), jnp.float32),
            pltpu.VMEM((NBUF, BLK, D), jnp.float32),
            pltpu.VMEM((BLK,), jnp.float32),
            pltpu.SemaphoreType.DMA((NBUF,)),
            pltpu.SemaphoreType.DMA((NBUF,)),
        ],
    )
    def _dot_sigmoid(t_hbm, c_hbm, out_hbm, t_buf, c_buf, s_buf, sem_t, sem_c):
        wid = lax.axis_index("s") * NC + lax.axis_index("c")
        base = wid * ROWS_PER_W

        def start(blk, slot):
            rows = pl.ds(base + blk * BLK, BLK)
            ht = pltpu.async_copy(t_hbm.at[rows], t_buf.at[slot], sem_t.at[slot])
            hc = pltpu.async_copy(c_hbm.at[rows], c_buf.at[slot], sem_c.at[slot])
            return ht, hc

        def compute(blk, slot):
            tb = t_buf.at[slot]
            cb = c_buf.at[slot]

            lane = lax.iota(jnp.int32, L)
            perms = [
                jnp.reshape((lane + sh) % L, (L, 1)) for sh in (8, 4, 2, 1)
            ]
            dnums = lax.GatherDimensionNumbers(
                offset_dims=(), collapsed_slice_dims=(0,), start_index_map=(0,)
            )

            def rot(x, p):
                return lax.gather(
                    x, p, dnums, slice_sizes=(1,),
                    mode=lax.GatherScatterMode.PROMISE_IN_BOUNDS,
                )

            def grp_body(g, carry):
                res = jnp.zeros((L,), jnp.float32)
                for j in range(L):
                    r = g * L + j
                    acc = tb[r, pl.ds(0, L)] * cb[r, pl.ds(0, L)]
                    for k in range(1, D // L):
                        acc = acc + tb[r, pl.ds(k * L, L)] * cb[r, pl.ds(k * L, L)]
                    for p in perms:
                        acc = acc + rot(acc, p)
                    res = jnp.where(lane == j, acc, res)
                s_buf[pl.ds(g * L, L)] = res
                return carry

            lax.fori_loop(0, BLK // L, grp_body, 0)

            def sig_body(v, carry):
                x = s_buf[pl.ds(v * L, L)]
                s_buf[pl.ds(v * L, L)] = 1.0 / (1.0 + jnp.exp(-x))
                return carry

            lax.fori_loop(0, BLK // L, sig_body, 0)
            pltpu.sync_copy(s_buf, out_hbm.at[pl.ds(base + blk * BLK, BLK)])

        handles = start(0, 0)
        for blk in range(NBLK):
            nxt = start(blk + 1, (blk + 1) % NBUF) if blk + 1 < NBLK else None
            handles[0].wait()
            handles[1].wait()
            compute(blk, blk % NBUF)
            handles = nxt

    return _dot_sigmoid


def kernel(target_embeds, context_embeds):
    return _build()(target_embeds, context_embeds)


# speedup vs baseline: 1.0146x; 1.0146x over previous
import functools

import jax
import jax.numpy as jnp
from jax import lax
from jax.experimental import pallas as pl
from jax.experimental.pallas import tpu as pltpu
from jax.experimental.pallas import tpu_sc as plsc

B = 16384
D = 128
L = 16
NC = 2
NS = 16
NW = NC * NS
ROWS_PER_W = B // NW
BLK = 128
NBLK = ROWS_PER_W // BLK
NBUF = 2


@functools.cache
def _build():
    mesh = plsc.VectorSubcoreMesh(
        core_axis_name="c", subcore_axis_name="s", num_cores=NC, num_subcores=NS
    )

    @functools.partial(
        pl.kernel,
        out_type=jax.ShapeDtypeStruct((B,), jnp.float32),
        mesh=mesh,
        scratch_types=[
            pltpu.VMEM((NBUF, BLK, D), jnp.float32),
            pltpu.VMEM((NBUF, BLK, D), jnp.float32),
            pltpu.VMEM((BLK,), jnp.float32),
            pltpu.SemaphoreType.DMA((NBUF,)),
            pltpu.SemaphoreType.DMA((NBUF,)),
        ],
    )
    def _dot_sigmoid(t_hbm, c_hbm, out_hbm, t_buf, c_buf, s_buf, sem_t, sem_c):
        wid = lax.axis_index("s") * NC + lax.axis_index("c")
        base = wid * ROWS_PER_W

        lane = lax.iota(jnp.int32, L)
        perms = [jnp.reshape((lane + sh) % L, (L, 1)) for sh in (8, 4, 2, 1)]
        dnums = lax.GatherDimensionNumbers(
            offset_dims=(), collapsed_slice_dims=(0,), start_index_map=(0,)
        )

        def rot(x, p):
            return lax.gather(
                x, p, dnums, slice_sizes=(1,),
                mode=lax.GatherScatterMode.PROMISE_IN_BOUNDS,
            )

        def start(blk, slot):
            rows = pl.ds(base + blk * BLK, BLK)
            pltpu.async_copy(t_hbm.at[rows], t_buf.at[slot], sem_t.at[slot])
            pltpu.async_copy(c_hbm.at[rows], c_buf.at[slot], sem_c.at[slot])

        def wait(slot):
            rows = pl.ds(base, BLK)
            pltpu.make_async_copy(
                t_hbm.at[rows], t_buf.at[slot], sem_t.at[slot]
            ).wait()
            pltpu.make_async_copy(
                c_hbm.at[rows], c_buf.at[slot], sem_c.at[slot]
            ).wait()

        def compute(blk, slot):
            tb = t_buf.at[slot]
            cb = c_buf.at[slot]

            @plsc.parallel_loop(0, BLK // L)
            def _grp(g):
                res = jnp.zeros((L,), jnp.float32)
                for j in range(L):
                    r = g * L + j
                    acc = tb[r, pl.ds(0, L)] * cb[r, pl.ds(0, L)]
                    for k in range(1, D // L):
                        acc = acc + tb[r, pl.ds(k * L, L)] * cb[r, pl.ds(k * L, L)]
                    for p in perms:
                        acc = acc + rot(acc, p)
                    res = jnp.where(lane == j, acc, res)
                s_buf[pl.ds(g * L, L)] = 1.0 / (1.0 + jnp.exp(-res))

            pltpu.sync_copy(s_buf, out_hbm.at[pl.ds(base + blk * BLK, BLK)])

        start(0, 0)

        def blk_body(blk, carry):
            slot = blk % NBUF

            @pl.when(blk + 1 < NBLK)
            def _():
                start(blk + 1, (blk + 1) % NBUF)

            wait(slot)
            compute(blk, slot)
            return carry

        lax.fori_loop(0, NBLK, blk_body, 0)

    return _dot_sigmoid


def kernel(target_embeds, context_embeds):
    return _build()(target_embeds, context_embeds)


# speedup vs baseline: 1.0167x; 1.0021x over previous
import functools

import jax
import jax.numpy as jnp
from jax import lax
from jax.experimental import pallas as pl
from jax.experimental.pallas import tpu as pltpu
from jax.experimental.pallas import tpu_sc as plsc

B = 16384
D = 128
L = 16
NC = 2
NS = 16
NW = NC * NS
ROWS_PER_W = B // NW
BLK = 128
NBLK = ROWS_PER_W // BLK
NBUF = 2


@functools.cache
def _build():
    mesh = plsc.VectorSubcoreMesh(
        core_axis_name="c", subcore_axis_name="s", num_cores=NC, num_subcores=NS
    )

    @functools.partial(
        pl.kernel,
        out_type=jax.ShapeDtypeStruct((B,), jnp.float32),
        mesh=mesh,
        scratch_types=[
            pltpu.VMEM((NBUF, BLK, D), jnp.float32),
            pltpu.VMEM((NBUF, BLK, D), jnp.float32),
            pltpu.VMEM((BLK,), jnp.float32),
            pltpu.SemaphoreType.DMA((NBUF,)),
            pltpu.SemaphoreType.DMA((NBUF,)),
        ],
    )
    def _dot_sigmoid(t_hbm, c_hbm, out_hbm, t_buf, c_buf, s_buf, sem_t, sem_c):
        wid = lax.axis_index("s") * NC + lax.axis_index("c")
        base = wid * ROWS_PER_W

        lane = lax.iota(jnp.int32, L)
        perms = [jnp.reshape((lane + sh) % L, (L, 1)) for sh in (8, 4, 2, 1)]
        dnums = lax.GatherDimensionNumbers(
            offset_dims=(), collapsed_slice_dims=(0,), start_index_map=(0,)
        )

        def rot(x, p):
            return lax.gather(
                x, p, dnums, slice_sizes=(1,),
                mode=lax.GatherScatterMode.PROMISE_IN_BOUNDS,
            )

        def start(blk, slot):
            rows = pl.ds(base + blk * BLK, BLK)
            pltpu.async_copy(t_hbm.at[rows], t_buf.at[slot], sem_t.at[slot])
            pltpu.async_copy(c_hbm.at[rows], c_buf.at[slot], sem_c.at[slot])

        def wait(slot):
            rows = pl.ds(base, BLK)
            pltpu.make_async_copy(
                t_hbm.at[rows], t_buf.at[slot], sem_t.at[slot]
            ).wait()
            pltpu.make_async_copy(
                c_hbm.at[rows], c_buf.at[slot], sem_c.at[slot]
            ).wait()

        def compute(blk, slot):
            tb = t_buf.at[slot]
            cb = c_buf.at[slot]

            @plsc.parallel_loop(0, BLK // L)
            def _grp(g):
                accs = [
                    tb[g * L + j, pl.ds(0, L)] * cb[g * L + j, pl.ds(0, L)]
                    for j in range(L)
                ]
                for k in range(1, D // L):
                    for j in range(L):
                        r = g * L + j
                        accs[j] = accs[j] + tb[r, pl.ds(k * L, L)] * cb[r, pl.ds(k * L, L)]
                res = jnp.zeros((L,), jnp.float32)
                for j in range(L):
                    acc = accs[j]
                    for p in perms:
                        acc = acc + rot(acc, p)
                    res = jnp.where(lane == j, acc, res)
                s_buf[pl.ds(g * L, L)] = 1.0 / (1.0 + jnp.exp(-res))

            pltpu.sync_copy(s_buf, out_hbm.at[pl.ds(base + blk * BLK, BLK)])

        start(0, 0)

        def step_body(step, carry):
            blk0 = step * NBUF

            @pl.when(blk0 + 1 < NBLK)
            def _():
                start(blk0 + 1, 1)

            wait(0)
            compute(blk0, 0)

            @pl.when(blk0 + 2 < NBLK)
            def _():
                start(blk0 + 2, 0)

            wait(1)
            compute(blk0 + 1, 1)
            return carry

        lax.fori_loop(0, NBLK // NBUF, step_body, 0)

    return _dot_sigmoid


def kernel(target_embeds, context_embeds):
    return _build()(target_embeds, context_embeds)
